# baseline (speedup 1.0000x reference)
import functools
import jax
import jax.numpy as jnp
from jax import lax
from jax.experimental import pallas as pl
from jax.experimental.pallas import tpu as pltpu
from jax.experimental.pallas import tpu_sc as plsc

N, E, D_V, D_E, D_A = 10000, 320000, 128, 16, 64

NC, NS, L = 2, 16, 16
NW = NC * NS
EPW = E // NW
CH = 80
NCH = EPW // CH
RPT = N // NS
BE = 2000
NBE = E // BE

_SC_MESH = plsc.VectorSubcoreMesh(
    core_axis_name="c", subcore_axis_name="s", num_cores=NC, num_subcores=NS)


def _qkv_body(nodes_ref, wq_ref, bq_ref, wk_ref, bk_ref, edges_ref, wv_ref,
              bv_ref, q_ref, k_ref, c_ref, v_ref):
    @pl.when(pl.program_id(0) == 0)
    def _():
        x = nodes_ref[...]
        q = jnp.dot(x, wq_ref[...],
                    preferred_element_type=jnp.float32) + bq_ref[...]
        q = jnp.where(q >= 0, q, 0.01 * q)
        k = jnp.dot(x, wk_ref[...],
                    preferred_element_type=jnp.float32) + bk_ref[...]
        k = jnp.where(k >= 0, k, 0.01 * k)
        q_ref[...] = q
        k_ref[...] = k
        qn = jnp.max(jnp.sum(q * q, axis=1))
        kn = jnp.max(jnp.sum(k * k, axis=1))
        c_ref[0, 0] = jnp.sqrt(qn * kn) * 0.125

    v = jnp.dot(edges_ref[...], wv_ref[...],
                preferred_element_type=jnp.float32) + bv_ref[...]
    v_ref[...] = jnp.where(v >= 0, v, 0.01 * v)


def _qkv_call(nodes, Wq, bq, Wk, bk, edges, Wv, bv):
    return pl.pallas_call(
        _qkv_body,
        grid=(NBE,),
        in_specs=[
            pl.BlockSpec((N, D_V), lambda i: (0, 0)),
            pl.BlockSpec((D_V, D_A), lambda i: (0, 0)),
            pl.BlockSpec((1, D_A), lambda i: (0, 0)),
            pl.BlockSpec((D_V, D_A), lambda i: (0, 0)),
            pl.BlockSpec((1, D_A), lambda i: (0, 0)),
            pl.BlockSpec((BE, D_E), lambda i: (i, 0)),
            pl.BlockSpec((D_E, D_A), lambda i: (0, 0)),
            pl.BlockSpec((1, D_A), lambda i: (0, 0)),
        ],
        out_shape=[
            jax.ShapeDtypeStruct((N, D_A), jnp.float32),
            jax.ShapeDtypeStruct((N, D_A), jnp.float32),
            jax.ShapeDtypeStruct((1, 1), jnp.float32),
            jax.ShapeDtypeStruct((E, D_A), jnp.float32),
        ],
        out_specs=[
            pl.BlockSpec((N, D_A), lambda i: (0, 0)),
            pl.BlockSpec((N, D_A), lambda i: (0, 0)),
            pl.BlockSpec(memory_space=pltpu.SMEM),
            pl.BlockSpec((BE, D_A), lambda i: (i, 0)),
        ],
    )(nodes, Wq, bq, Wk, bk, edges, Wv, bv)


def _logit_body(q_hbm, kn_hbm, s_hbm, r_hbm, c_hbm, ex_hbm, denp_hbm,
                sidx_v, ridx_v, qa_v, ka_v, qb_v, kb_v, ex_v, den_v, c_v,
                sema, semb):
    wid = lax.axis_index("s") * NC + lax.axis_index("c")
    pltpu.sync_copy(c_hbm, c_v)
    cvec = c_v[...]
    pltpu.sync_copy(s_hbm.at[wid], sidx_v)
    pltpu.sync_copy(r_hbm.at[wid], ridx_v)

    def zden(i, carry):
        den_v[pl.ds(i * L, L)] = jnp.zeros((L,), jnp.float32)
        return carry
    lax.fori_loop(0, N // L, zden, 0)

    def issue(c, qbuf, kbuf, sem):
        pltpu.async_copy(q_hbm.at[ridx_v.at[c]], qbuf, sem)
        pltpu.async_copy(kn_hbm.at[sidx_v.at[c]], kbuf, sem)

    def drain(qbuf, kbuf, sem):
        pltpu.make_async_copy(q_hbm.at[ridx_v.at[0]], qbuf, sem).wait()
        pltpu.make_async_copy(kn_hbm.at[sidx_v.at[0]], kbuf, sem).wait()

    def compute(c, qbuf, kbuf):
        for g in range(CH // L):
            row16 = lax.iota(jnp.int32, L) + (g * L)
            acc = jnp.zeros((L,), jnp.float32)
            for d in range(D_A):
                col16 = (lax.iota(jnp.int32, L) + d) & (D_A - 1)
                qv = plsc.load_gather(qbuf, [row16, col16])
                kv = plsc.load_gather(kbuf, [row16, col16])
                acc = acc + qv * kv
            ex16 = jnp.exp(acc * 0.125 - cvec)
            ex_v[pl.ds(c * CH + g * L, L)] = ex16
            r16 = ridx_v[c, pl.ds(g * L, L)]
            plsc.addupdate_scatter(den_v, [r16], ex16)

    issue(0, qa_v, ka_v, sema)

    def body(j, carry):
        c0 = 2 * j
        issue(c0 + 1, qb_v, kb_v, semb)
        drain(qa_v, ka_v, sema)
        compute(c0, qa_v, ka_v)
        issue(c0 + 2, qa_v, ka_v, sema)
        drain(qb_v, kb_v, semb)
        compute(c0 + 1, qb_v, kb_v)
        return carry

    lax.fori_loop(0, (NCH - 1) // 2, body, 0)
    drain(qa_v, ka_v, sema)
    compute(NCH - 1, qa_v, ka_v)

    pltpu.sync_copy(ex_v, ex_hbm.at[pl.ds(wid * EPW, EPW)])
    pltpu.sync_copy(den_v, denp_hbm.at[pl.ds(wid * N, N)])


_logit_call = functools.partial(
    pl.kernel,
    _logit_body,
    out_type=[
        jax.ShapeDtypeStruct((E,), jnp.float32),
        jax.ShapeDtypeStruct((NW * N,), jnp.float32),
    ],
    mesh=_SC_MESH,
    scratch_types=[
        pltpu.VMEM((NCH, CH), jnp.int32),
        pltpu.VMEM((NCH, CH), jnp.int32),
        pltpu.VMEM((CH, D_A), jnp.float32),
        pltpu.VMEM((CH, D_A), jnp.float32),
        pltpu.VMEM((CH, D_A), jnp.float32),
        pltpu.VMEM((CH, D_A), jnp.float32),
        pltpu.VMEM((EPW,), jnp.float32),
        pltpu.VMEM((N,), jnp.float32),
        pltpu.VMEM((L,), jnp.float32),
        pltpu.SemaphoreType.DMA,
        pltpu.SemaphoreType.DMA,
    ],
    compiler_params=pltpu.CompilerParams(
        use_tc_tiling_on_sc=False, needs_layout_passes=False),
)()


def _scatter_body(v_hbm, ex_hbm, r_hbm, z_hbm, up_hbm, ridx_v, exs_v,
                  ra_v, rb_v, u_sh, sema, semb, semsa, semsb):
    cid = lax.axis_index("c")
    sid = lax.axis_index("s")
    wid = sid * NC + cid
    base0 = wid * EPW

    pltpu.sync_copy(r_hbm.at[wid], ridx_v)
    pltpu.sync_copy(ex_hbm.at[wid], exs_v)

    @pl.when(sid < NS - 1)
    def _():
        pltpu.sync_copy(z_hbm, u_sh.at[pl.ds(sid * 640, 640)])

    @pl.when(sid == NS - 1)
    def _():
        pltpu.sync_copy(z_hbm.at[pl.ds(0, 400)],
                        u_sh.at[pl.ds(sid * 640, 400)])

    plsc.subcore_barrier()

    def issue(c, buf, sem):
        pltpu.async_copy(v_hbm.at[pl.ds(base0 + c * CH, CH)], buf, sem)

    def drain_load(buf, sem):
        pltpu.make_async_copy(v_hbm.at[pl.ds(0, CH)], buf, sem).wait()

    def scale(c, buf):
        for g in range(CH // L):
            ex16 = exs_v[c, pl.ds(g * L, L)]
            for l in range(L):
                e = g * L + l
                sc = lax.gather(
                    ex16, jnp.full((L, 1), l, jnp.int32),
                    lax.GatherDimensionNumbers(
                        offset_dims=(), collapsed_slice_dims=(0,),
                        start_index_map=(0,)),
                    slice_sizes=(1,),
                    mode=lax.GatherScatterMode.PROMISE_IN_BOUNDS)
                for j in range(D_A // L):
                    buf[e, pl.ds(j * L, L)] = buf[e, pl.ds(j * L, L)] * sc

    def issue_add(c, buf, sem):
        pltpu.async_copy(buf, u_sh.at[ridx_v.at[c]], sem, add=True)

    def drain_add(buf, sem):
        pltpu.make_async_copy(buf, u_sh.at[ridx_v.at[0]], sem).wait()

    issue(0, ra_v, sema)
    issue(1, rb_v, semb)

    def body(j, carry):
        c0 = 2 * j
        drain_load(ra_v, sema)
        scale(c0, ra_v)
        issue_add(c0, ra_v, semsa)
        drain_load(rb_v, semb)
        scale(c0 + 1, rb_v)
        issue_add(c0 + 1, rb_v, semsb)
        drain_add(ra_v, semsa)
        issue(c0 + 2, ra_v, sema)
        drain_add(rb_v, semsb)

        @pl.when(c0 + 3 < NCH)
        def _():
            issue(c0 + 3, rb_v, semb)
        return carry

    lax.fori_loop(0, (NCH - 1) // 2, body, 0)
    drain_load(ra_v, sema)
    scale(NCH - 1, ra_v)
    pltpu.sync_copy(ra_v, u_sh.at[ridx_v.at[NCH - 1]], add=True)
    plsc.subcore_barrier()

    @pl.when(sid < NS - 1)
    def _():
        pltpu.sync_copy(u_sh.at[pl.ds(sid * 640, 640)],
                        up_hbm.at[cid, pl.ds(sid * 640, 640)])

    @pl.when(sid == NS - 1)
    def _():
        pltpu.sync_copy(u_sh.at[pl.ds(sid * 640, 400)],
                        up_hbm.at[cid, pl.ds(sid * 640, 400)])


_scatter_call = functools.partial(
    pl.kernel,
    _scatter_body,
    out_type=jax.ShapeDtypeStruct((NC, N, D_A), jnp.float32),
    mesh=_SC_MESH,
    scratch_types=[
        pltpu.VMEM((NCH, CH), jnp.int32),
        pltpu.VMEM((NCH, CH), jnp.float32),
        pltpu.VMEM((CH, D_A), jnp.float32),
        pltpu.VMEM((CH, D_A), jnp.float32),
        pltpu.VMEM_SHARED((N, D_A), jnp.float32),
        pltpu.SemaphoreType.DMA,
        pltpu.SemaphoreType.DMA,
        pltpu.SemaphoreType.DMA,
        pltpu.SemaphoreType.DMA,
    ],
    compiler_params=pltpu.CompilerParams(
        use_tc_tiling_on_sc=False, needs_layout_passes=False),
)()


def _fin_body(denp_ref, up_ref, out_ref):
    den = jnp.sum(denp_ref[...], axis=0)
    rd = jnp.where(den > 0, 1.0 / den, 1.0)
    u = jnp.sum(up_ref[...], axis=0)
    out_ref[...] = u * rd[:, None]


def _fin_call(denp, up):
    return pl.pallas_call(
        _fin_body,
        out_shape=jax.ShapeDtypeStruct((N, D_A), jnp.float32),
    )(denp, up)


def kernel(nodes, edges, edge_index, Wq, bq, Wk, bk, Wv, bv):
    s3 = edge_index[0].astype(jnp.int32).reshape(NW, NCH, CH)
    r3 = edge_index[1].astype(jnp.int32).reshape(NW, NCH, CH)
    q, kn, c, v = _qkv_call(nodes, Wq, bq.reshape(1, D_A), Wk,
                            bk.reshape(1, D_A), edges, Wv, bv.reshape(1, D_A))
    c16 = jnp.broadcast_to(c.reshape(()), (L,))
    ex, denp_raw = _logit_call(q, kn, s3, r3, c16)
    denp = denp_raw.reshape(NW, N)
    ex3 = ex.reshape(NW, NCH, CH)
    zrows = jnp.zeros((640, D_A), jnp.float32)
    up = _scatter_call(v, ex3, r3, zrows)
    return _fin_call(denp, up)


# speedup vs baseline: 1.1890x; 1.1890x over previous
import functools
import jax
import jax.numpy as jnp
from jax import lax
from jax.experimental import pallas as pl
from jax.experimental.pallas import tpu as pltpu
from jax.experimental.pallas import tpu_sc as plsc

N, E, D_V, D_E, D_A = 10000, 320000, 128, 16, 64

NC, NS, L = 2, 16, 16
NW = NC * NS
EPW = E // NW
CH = 80
NCH = EPW // CH
RPT = N // NS
BE = 2000
NBE = E // BE

_SC_MESH = plsc.VectorSubcoreMesh(
    core_axis_name="c", subcore_axis_name="s", num_cores=NC, num_subcores=NS)


def _qk_body(nodes_ref, wq_ref, bq_ref, wk_ref, bk_ref, q_ref, k_ref, c_ref):
    x = nodes_ref[...]
    q = jnp.dot(x, wq_ref[...], preferred_element_type=jnp.float32) + bq_ref[...]
    q = jnp.where(q >= 0, q, 0.01 * q)
    k = jnp.dot(x, wk_ref[...], preferred_element_type=jnp.float32) + bk_ref[...]
    k = jnp.where(k >= 0, k, 0.01 * k)
    q_ref[...] = q
    k_ref[...] = k
    qn = jnp.max(jnp.sum(q * q, axis=1))
    kn = jnp.max(jnp.sum(k * k, axis=1))
    c_ref[0, 0] = jnp.sqrt(qn * kn) * 0.125


def _qk_call(nodes, Wq, bq, Wk, bk):
    return pl.pallas_call(
        _qk_body,
        out_shape=[
            jax.ShapeDtypeStruct((N, D_A), jnp.float32),
            jax.ShapeDtypeStruct((N, D_A), jnp.float32),
            jax.ShapeDtypeStruct((1, 1), jnp.float32),
        ],
        out_specs=[
            pl.BlockSpec((N, D_A), lambda: (0, 0)),
            pl.BlockSpec((N, D_A), lambda: (0, 0)),
            pl.BlockSpec(memory_space=pltpu.SMEM),
        ],
    )(nodes, Wq, bq, Wk, bk)


def _logit_body(q_hbm, kn_hbm, s_hbm, r_hbm, c_hbm, ex_hbm, denp_hbm,
                sidx_v, ridx_v, qa_v, ka_v, qb_v, kb_v, ex_v, den_v, c_v,
                sema, semb):
    wid = lax.axis_index("s") * NC + lax.axis_index("c")
    pltpu.sync_copy(c_hbm, c_v)
    cvec = c_v[...]
    pltpu.sync_copy(s_hbm.at[wid], sidx_v)
    pltpu.sync_copy(r_hbm.at[wid], ridx_v)

    def zden(i, carry):
        den_v[pl.ds(i * L, L)] = jnp.zeros((L,), jnp.float32)
        return carry
    lax.fori_loop(0, N // L, zden, 0)

    def issue(c, qbuf, kbuf, sem):
        pltpu.async_copy(q_hbm.at[ridx_v.at[c]], qbuf, sem)
        pltpu.async_copy(kn_hbm.at[sidx_v.at[c]], kbuf, sem)

    def drain(qbuf, kbuf, sem):
        pltpu.make_async_copy(q_hbm.at[ridx_v.at[0]], qbuf, sem).wait()
        pltpu.make_async_copy(kn_hbm.at[sidx_v.at[0]], kbuf, sem).wait()

    def compute(c, qbuf, kbuf):
        for g in range(CH // L):
            row16 = lax.iota(jnp.int32, L) + (g * L)
            acc = jnp.zeros((L,), jnp.float32)
            for d in range(D_A):
                col16 = (lax.iota(jnp.int32, L) + d) & (D_A - 1)
                qv = plsc.load_gather(qbuf, [row16, col16])
                kv = plsc.load_gather(kbuf, [row16, col16])
                acc = acc + qv * kv
            ex16 = jnp.exp(acc * 0.125 - cvec)
            ex_v[pl.ds(c * CH + g * L, L)] = ex16
            r16 = ridx_v[c, pl.ds(g * L, L)]
            plsc.addupdate_scatter(den_v, [r16], ex16)

    issue(0, qa_v, ka_v, sema)

    def body(j, carry):
        c0 = 2 * j
        issue(c0 + 1, qb_v, kb_v, semb)
        drain(qa_v, ka_v, sema)
        compute(c0, qa_v, ka_v)
        issue(c0 + 2, qa_v, ka_v, sema)
        drain(qb_v, kb_v, semb)
        compute(c0 + 1, qb_v, kb_v)
        return carry

    lax.fori_loop(0, (NCH - 1) // 2, body, 0)
    drain(qa_v, ka_v, sema)
    compute(NCH - 1, qa_v, ka_v)

    pltpu.sync_copy(ex_v, ex_hbm.at[pl.ds(wid * EPW, EPW)])
    pltpu.sync_copy(den_v, denp_hbm.at[pl.ds(wid * N, N)])


_logit_call = functools.partial(
    pl.kernel,
    _logit_body,
    out_type=[
        jax.ShapeDtypeStruct((E,), jnp.float32),
        jax.ShapeDtypeStruct((NW * N,), jnp.float32),
    ],
    mesh=_SC_MESH,
    scratch_types=[
        pltpu.VMEM((NCH, CH), jnp.int32),
        pltpu.VMEM((NCH, CH), jnp.int32),
        pltpu.VMEM((CH, D_A), jnp.float32),
        pltpu.VMEM((CH, D_A), jnp.float32),
        pltpu.VMEM((CH, D_A), jnp.float32),
        pltpu.VMEM((CH, D_A), jnp.float32),
        pltpu.VMEM((EPW,), jnp.float32),
        pltpu.VMEM((N,), jnp.float32),
        pltpu.VMEM((L,), jnp.float32),
        pltpu.SemaphoreType.DMA,
        pltpu.SemaphoreType.DMA,
    ],
    compiler_params=pltpu.CompilerParams(
        use_tc_tiling_on_sc=False, needs_layout_passes=False),
)()


def _edge_body(edges_ref, wv_ref, bv_ref, v_ref):
    v = jnp.dot(edges_ref[...], wv_ref[...], preferred_element_type=jnp.float32)
    v = v + bv_ref[...]
    v_ref[...] = jnp.where(v >= 0, v, 0.01 * v)


def _edge_call(edges, Wv, bv):
    return pl.pallas_call(
        _edge_body,
        grid=(NBE,),
        in_specs=[
            pl.BlockSpec((BE, D_E), lambda i: (i, 0)),
            pl.BlockSpec((D_E, D_A), lambda i: (0, 0)),
            pl.BlockSpec((1, D_A), lambda i: (0, 0)),
        ],
        out_specs=pl.BlockSpec((BE, D_A), lambda i: (i, 0)),
        out_shape=jax.ShapeDtypeStruct((E, D_A), jnp.float32),
    )(edges, Wv, bv)


def _scatter_body(v_hbm, ex_hbm, r_hbm, z_hbm, up_hbm, ridx_v, exs_v,
                  ra_v, rb_v, u_sh, sema, semb, semsa, semsb):
    cid = lax.axis_index("c")
    sid = lax.axis_index("s")
    wid = sid * NC + cid
    base0 = wid * EPW

    pltpu.sync_copy(r_hbm.at[wid], ridx_v)
    pltpu.sync_copy(ex_hbm.at[wid], exs_v)

    @pl.when(sid < NS - 1)
    def _():
        pltpu.sync_copy(z_hbm, u_sh.at[pl.ds(sid * 640, 640)])

    @pl.when(sid == NS - 1)
    def _():
        pltpu.sync_copy(z_hbm.at[pl.ds(0, 400)],
                        u_sh.at[pl.ds(sid * 640, 400)])

    plsc.subcore_barrier()

    def issue(c, buf, sem):
        pltpu.async_copy(v_hbm.at[pl.ds(base0 + c * CH, CH)], buf, sem)

    def drain_load(buf, sem):
        pltpu.make_async_copy(v_hbm.at[pl.ds(0, CH)], buf, sem).wait()

    def scale(c, buf):
        for g in range(CH // L):
            ex16 = exs_v[c, pl.ds(g * L, L)]
            for l in range(L):
                e = g * L + l
                sc = lax.gather(
                    ex16, jnp.full((L, 1), l, jnp.int32),
                    lax.GatherDimensionNumbers(
                        offset_dims=(), collapsed_slice_dims=(0,),
                        start_index_map=(0,)),
                    slice_sizes=(1,),
                    mode=lax.GatherScatterMode.PROMISE_IN_BOUNDS)
                for j in range(D_A // L):
                    buf[e, pl.ds(j * L, L)] = buf[e, pl.ds(j * L, L)] * sc

    def issue_add(c, buf, sem):
        pltpu.async_copy(buf, u_sh.at[ridx_v.at[c]], sem, add=True)

    def drain_add(buf, sem):
        pltpu.make_async_copy(buf, u_sh.at[ridx_v.at[0]], sem).wait()

    issue(0, ra_v, sema)
    issue(1, rb_v, semb)

    def body(j, carry):
        c0 = 2 * j
        drain_load(ra_v, sema)
        scale(c0, ra_v)
        issue_add(c0, ra_v, semsa)
        drain_load(rb_v, semb)
        scale(c0 + 1, rb_v)
        issue_add(c0 + 1, rb_v, semsb)
        drain_add(ra_v, semsa)
        issue(c0 + 2, ra_v, sema)
        drain_add(rb_v, semsb)

        @pl.when(c0 + 3 < NCH)
        def _():
            issue(c0 + 3, rb_v, semb)
        return carry

    lax.fori_loop(0, (NCH - 1) // 2, body, 0)
    drain_load(ra_v, sema)
    scale(NCH - 1, ra_v)
    pltpu.sync_copy(ra_v, u_sh.at[ridx_v.at[NCH - 1]], add=True)
    plsc.subcore_barrier()

    @pl.when(sid < NS - 1)
    def _():
        pltpu.sync_copy(u_sh.at[pl.ds(sid * 640, 640)],
                        up_hbm.at[cid, pl.ds(sid * 640, 640)])

    @pl.when(sid == NS - 1)
    def _():
        pltpu.sync_copy(u_sh.at[pl.ds(sid * 640, 400)],
                        up_hbm.at[cid, pl.ds(sid * 640, 400)])


_scatter_call = functools.partial(
    pl.kernel,
    _scatter_body,
    out_type=jax.ShapeDtypeStruct((NC, N, D_A), jnp.float32),
    mesh=_SC_MESH,
    scratch_types=[
        pltpu.VMEM((NCH, CH), jnp.int32),
        pltpu.VMEM((NCH, CH), jnp.float32),
        pltpu.VMEM((CH, D_A), jnp.float32),
        pltpu.VMEM((CH, D_A), jnp.float32),
        pltpu.VMEM_SHARED((N, D_A), jnp.float32),
        pltpu.SemaphoreType.DMA,
        pltpu.SemaphoreType.DMA,
        pltpu.SemaphoreType.DMA,
        pltpu.SemaphoreType.DMA,
    ],
    compiler_params=pltpu.CompilerParams(
        use_tc_tiling_on_sc=False, needs_layout_passes=False),
)()


def _fin_body(denp_ref, up_ref, out_ref):
    den = jnp.sum(denp_ref[...], axis=0)
    rd = jnp.where(den > 0, 1.0 / den, 1.0)
    u = jnp.sum(up_ref[...], axis=0)
    out_ref[...] = u * rd[:, None]


def _fin_call(denp, up):
    return pl.pallas_call(
        _fin_body,
        out_shape=jax.ShapeDtypeStruct((N, D_A), jnp.float32),
    )(denp, up)


def kernel(nodes, edges, edge_index, Wq, bq, Wk, bk, Wv, bv):
    s3 = edge_index[0].astype(jnp.int32).reshape(NW, NCH, CH)
    r3 = edge_index[1].astype(jnp.int32).reshape(NW, NCH, CH)
    q, kn, c = _qk_call(nodes, Wq, bq.reshape(1, D_A), Wk, bk.reshape(1, D_A))
    v = _edge_call(edges, Wv, bv.reshape(1, D_A))
    c16 = jnp.broadcast_to(c.reshape(()), (L,))
    ex, denp_raw = _logit_call(q, kn, s3, r3, c16)
    denp = denp_raw.reshape(NW, N)
    ex3 = ex.reshape(NW, NCH, CH)
    zrows = jnp.zeros((640, D_A), jnp.float32)
    up = _scatter_call(v, ex3, r3, zrows)
    return _fin_call(denp, up)
